# baseline (speedup 1.0000x reference)
import functools

import jax
import jax.numpy as jnp
from jax import lax
from jax.experimental import pallas as pl
from jax.experimental.pallas import tpu as pltpu
from jax.experimental.pallas import tpu_sc as plsc

N_ATOMS = 10000
MAX_NEI = 32
HIDDEN = 128
ATOM_FDIM = 133
BOND_FDIM = 14
DEPTH = 3

NW = 32
APW = 320
N_PAD = NW * APW
CHUNK_ATOMS = 4
CHUNKS = APW // CHUNK_ATOMS
_NBUF = 4
_MGROUPS = HIDDEN // 32


def _gelu_exact(x):
    return 0.5 * x * (1.0 + lax.erf(x * 0.7071067811865476))


def _pipelined_rounds(gcopy, ocopy, nchunks, reduce_chunk):
    for b in range(_NBUF - 1):
        gcopy(b, b).start()

    def quad_body(i, _):
        k0 = i * _NBUF
        for b in range(_NBUF):
            k = k0 + b

            @pl.when(k + _NBUF - 1 < nchunks)
            def _():
                gcopy(k + _NBUF - 1, (b + _NBUF - 1) % _NBUF).start()

            gcopy(k, b).wait()

            @pl.when(k >= _NBUF)
            def _():
                ocopy(k - _NBUF, b).wait()

            reduce_chunk(b)
            ocopy(k, b).start()
        return 0

    lax.fori_loop(0, nchunks // _NBUF, quad_body, 0)
    for b in range(_NBUF):
        ocopy(nchunks - _NBUF + b, b).wait()


@functools.partial(
    pl.kernel,
    out_type=jax.ShapeDtypeStruct((N_PAD, HIDDEN), jnp.bfloat16),
    mesh=plsc.VectorSubcoreMesh(core_axis_name="c", subcore_axis_name="s"),
    compiler_params=pltpu.CompilerParams(use_tc_tiling_on_sc=False,
                                         needs_layout_passes=False),
    scratch_types=[
        pltpu.VMEM((CHUNKS, 128), jnp.int32),
        pltpu.VMEM((_NBUF, 128, HIDDEN), jnp.bfloat16),
        pltpu.VMEM((_NBUF, CHUNK_ATOMS, HIDDEN), jnp.bfloat16),
        pltpu.VMEM_SHARED((N_ATOMS, HIDDEN), jnp.bfloat16),
    ] + [pltpu.SemaphoreType.DMA] * (2 * _NBUF),
)
def _round_msg(mtab, idx_hbm, out_hbm, idx_v, rows_v, out_v, shared, *sems):
    gsems, osems = sems[:_NBUF], sems[_NBUF:]
    wid = lax.axis_index("s") * 2 + lax.axis_index("c")
    pltpu.sync_copy(idx_hbm.at[wid], idx_v)

    sub = lax.axis_index("s")
    rpw = N_ATOMS // 16
    pltpu.sync_copy(mtab.at[pl.ds(sub * rpw, rpw)],
                    shared.at[pl.ds(sub * rpw, rpw)])
    plsc.subcore_barrier()

    def gcopy(k, b):
        return pltpu.make_async_copy(
            shared.at[idx_v.at[k]], rows_v.at[b], gsems[b])

    def ocopy(k, b):
        return pltpu.make_async_copy(
            out_v.at[b],
            out_hbm.at[pl.ds(wid * APW + k * CHUNK_ATOMS, CHUNK_ATOMS)],
            osems[b])

    def reduce_chunk(b):
        for a in range(CHUNK_ATOMS):
            f32accs = [None] * (2 * _MGROUPS)
            for r in range(MAX_NEI):
                for g in range(_MGROUPS):
                    lo, hi = plsc.unpack(
                        rows_v[b, a * MAX_NEI + r, pl.ds(32 * g, 32)],
                        format=plsc.PackFormat.INTERLEAVED)
                    if f32accs[2 * g] is None:
                        f32accs[2 * g], f32accs[2 * g + 1] = lo, hi
                    else:
                        f32accs[2 * g] = f32accs[2 * g] + lo
                        f32accs[2 * g + 1] = f32accs[2 * g + 1] + hi
            for g in range(_MGROUPS):
                out_v[b, a, pl.ds(32 * g, 32)] = plsc.pack(
                    f32accs[2 * g], f32accs[2 * g + 1],
                    format=plsc.PackFormat.INTERLEAVED)

    _pipelined_rounds(gcopy, ocopy, CHUNKS, reduce_chunk)


@functools.partial(
    pl.kernel,
    out_type=jax.ShapeDtypeStruct((N_PAD, 16), jnp.float32),
    mesh=plsc.VectorSubcoreMesh(core_axis_name="c", subcore_axis_name="s"),
    compiler_params=pltpu.CompilerParams(use_tc_tiling_on_sc=False,
                                         needs_layout_passes=False),
    scratch_types=[
        pltpu.VMEM((CHUNKS, 128), jnp.int32),
        pltpu.VMEM((_NBUF, 128, 16), jnp.float32),
        pltpu.VMEM((_NBUF, CHUNK_ATOMS, 16), jnp.float32),
    ] + [pltpu.SemaphoreType.DMA] * (2 * _NBUF),
)
def _round_bond(btab, idx_hbm, out_hbm, idx_v, rows_v, out_v, *sems):
    gsems, osems = sems[:_NBUF], sems[_NBUF:]
    wid = lax.axis_index("s") * 2 + lax.axis_index("c")
    pltpu.sync_copy(idx_hbm.at[wid], idx_v)

    def gcopy(k, b):
        return pltpu.make_async_copy(
            btab.at[idx_v.at[k]], rows_v.at[b], gsems[b])

    def ocopy(k, b):
        return pltpu.make_async_copy(
            out_v.at[b],
            out_hbm.at[pl.ds(wid * APW + k * CHUNK_ATOMS, CHUNK_ATOMS)],
            osems[b])

    def reduce_chunk(b):
        for a in range(CHUNK_ATOMS):
            acc = rows_v[b, a * MAX_NEI, pl.ds(0, 16)]
            for r in range(1, MAX_NEI):
                acc = acc + rows_v[b, a * MAX_NEI + r, pl.ds(0, 16)]
            out_v[b, a, pl.ds(0, 16)] = acc

    _pipelined_rounds(gcopy, ocopy, CHUNKS, reduce_chunk)


_ROWS = 2000
_GRID = N_ATOMS // _ROWS


def _row_mask(pid, x):
    rows = lax.broadcasted_iota(jnp.int32, x.shape, 0) + pid * _ROWS
    return jnp.where(rows == 0, 0.0, x)


def _prologue_body(x_ref, wi_ref, w0_ref, w1_ref, w2_ref,
                   inp_ref, inpb_ref, h_ref):
    pid = pl.program_id(0)
    x = x_ref[...]
    inp = jnp.dot(x, wi_ref[...], preferred_element_type=jnp.float32)
    inp = _row_mask(pid, inp)
    inp_ref[...] = inp
    inpb_ref[...] = inp.astype(jnp.bfloat16)
    h = _gelu_exact(jnp.dot(x, w0_ref[...], preferred_element_type=jnp.float32))
    h = _gelu_exact(jnp.dot(h, w1_ref[...], preferred_element_type=jnp.float32))
    h = _gelu_exact(jnp.dot(h, w2_ref[...], preferred_element_type=jnp.float32))
    h_ref[...] = h


def _tc_prologue(f_atoms, W_i, W0, W1, W2):
    return pl.pallas_call(
        _prologue_body,
        grid=(_GRID,),
        in_specs=[
            pl.BlockSpec((_ROWS, ATOM_FDIM), lambda i: (i, 0)),
            pl.BlockSpec((ATOM_FDIM, HIDDEN), lambda i: (0, 0)),
            pl.BlockSpec((ATOM_FDIM, HIDDEN), lambda i: (0, 0)),
            pl.BlockSpec((HIDDEN, HIDDEN), lambda i: (0, 0)),
            pl.BlockSpec((HIDDEN, HIDDEN), lambda i: (0, 0)),
        ],
        out_specs=[
            pl.BlockSpec((_ROWS, HIDDEN), lambda i: (i, 0)),
            pl.BlockSpec((_ROWS, HIDDEN), lambda i: (i, 0)),
            pl.BlockSpec((_ROWS, HIDDEN), lambda i: (i, 0)),
        ],
        out_shape=[
            jax.ShapeDtypeStruct((N_ATOMS, HIDDEN), jnp.float32),
            jax.ShapeDtypeStruct((N_ATOMS, HIDDEN), jnp.bfloat16),
            jax.ShapeDtypeStruct((N_ATOMS, HIDDEN), jnp.float32),
        ],
    )(f_atoms, W_i, W0, W1, W2)


def _update_body(m_ref, s_ref, b_ref, wt_ref, wb_ref, o_ref, ob_ref):
    pid = pl.program_id(0)
    s = s_ref[...].astype(jnp.float32)
    m = (m_ref[...]
         + jnp.dot(s, wt_ref[...], preferred_element_type=jnp.float32)
         + jnp.dot(b_ref[...], wb_ref[...], preferred_element_type=jnp.float32))
    m = _row_mask(pid, m)
    o_ref[...] = m
    ob_ref[...] = m.astype(jnp.bfloat16)


def _tc_update(message, s, sumb, Wh_top, Wh_bot16):
    return pl.pallas_call(
        _update_body,
        grid=(_GRID,),
        in_specs=[
            pl.BlockSpec((_ROWS, HIDDEN), lambda i: (i, 0)),
            pl.BlockSpec((_ROWS, HIDDEN), lambda i: (i, 0)),
            pl.BlockSpec((_ROWS, 16), lambda i: (i, 0)),
            pl.BlockSpec((HIDDEN, HIDDEN), lambda i: (0, 0)),
            pl.BlockSpec((16, HIDDEN), lambda i: (0, 0)),
        ],
        out_specs=[
            pl.BlockSpec((_ROWS, HIDDEN), lambda i: (i, 0)),
            pl.BlockSpec((_ROWS, HIDDEN), lambda i: (i, 0)),
        ],
        out_shape=[
            jax.ShapeDtypeStruct((N_ATOMS, HIDDEN), jnp.float32),
            jax.ShapeDtypeStruct((N_ATOMS, HIDDEN), jnp.bfloat16),
        ],
    )(message, s, sumb, Wh_top, Wh_bot16)


def _final_body(h_ref, s_ref, wt_ref, wb_ref, o_ref):
    s = s_ref[...].astype(jnp.float32)
    o = (jnp.dot(h_ref[...], wt_ref[...], preferred_element_type=jnp.float32)
         + jnp.dot(s, wb_ref[...], preferred_element_type=jnp.float32))
    o_ref[...] = _gelu_exact(o)


def _tc_final(h, s, Wo_top, Wo_bot):
    return pl.pallas_call(
        _final_body,
        grid=(_GRID,),
        in_specs=[
            pl.BlockSpec((_ROWS, HIDDEN), lambda i: (i, 0)),
            pl.BlockSpec((_ROWS, HIDDEN), lambda i: (i, 0)),
            pl.BlockSpec((HIDDEN, HIDDEN), lambda i: (0, 0)),
            pl.BlockSpec((HIDDEN, HIDDEN), lambda i: (0, 0)),
        ],
        out_specs=pl.BlockSpec((_ROWS, HIDDEN), lambda i: (i, 0)),
        out_shape=jax.ShapeDtypeStruct((N_ATOMS, HIDDEN), jnp.float32),
    )(h, s, Wo_top, Wo_bot)


_BROWS = 4000
_BGRID = 320000 // _BROWS


def _pad_body(b_ref, o_ref):
    o_ref[...] = jnp.concatenate(
        [b_ref[...], jnp.zeros((_BROWS, 16 - BOND_FDIM), jnp.float32)], axis=1)


def _tc_pad_bonds(f_bonds):
    return pl.pallas_call(
        _pad_body,
        grid=(_BGRID,),
        in_specs=[pl.BlockSpec((_BROWS, BOND_FDIM), lambda i: (i, 0))],
        out_specs=pl.BlockSpec((_BROWS, 16), lambda i: (i, 0)),
        out_shape=jax.ShapeDtypeStruct((320000, 16), jnp.float32),
    )(f_bonds)


def _pack_idx(idx):
    idx = jnp.pad(idx.astype(jnp.int32), ((0, N_PAD - N_ATOMS), (0, 0)))
    return idx.reshape(NW, CHUNKS, 128)


def kernel(f_atoms, f_bonds, a2a, a2b, W_i, W_ah0, W_ah1, W_ah2,
           W_h0, W_h1, W_h2, W_o):
    idx_a = _pack_idx(a2a)
    idx_b = _pack_idx(a2b)
    f_bonds16 = _tc_pad_bonds(f_bonds)

    W_h = [W_h0, W_h1, W_h2]
    Wh_top = [w[:HIDDEN] for w in W_h]
    Wh_bot16 = [jnp.pad(w[HIDDEN:], ((0, 2), (0, 0))) for w in W_h]

    inp, inp_bf, h = _tc_prologue(f_atoms, W_i, W_ah0, W_ah1, W_ah2)
    sumb = _round_bond(f_bonds16, idx_b)[:N_ATOMS]

    message, message_bf = inp, inp_bf
    for d in range(DEPTH):
        s = _round_msg(message_bf, idx_a)[:N_ATOMS]
        message, message_bf = _tc_update(message, s, sumb,
                                         Wh_top[d], Wh_bot16[d])

    s = _round_msg(message_bf, idx_a)[:N_ATOMS]
    return _tc_final(h, s, W_o[:HIDDEN], W_o[HIDDEN:])


# speedup vs baseline: 1.0980x; 1.0980x over previous
import functools

import jax
import jax.numpy as jnp
from jax import lax
from jax.experimental import pallas as pl
from jax.experimental.pallas import tpu as pltpu
from jax.experimental.pallas import tpu_sc as plsc

N_ATOMS = 10000
MAX_NEI = 32
HIDDEN = 128
ATOM_FDIM = 133
BOND_FDIM = 14
DEPTH = 3

NW = 32
APW = 320
N_PAD = NW * APW
CHUNK_ATOMS = 4
CHUNKS = APW // CHUNK_ATOMS
_NBUF = 4
_MGROUPS = HIDDEN // 32


def _gelu_exact(x):
    return 0.5 * x * (1.0 + lax.erf(x * 0.7071067811865476))


def _pipelined_rounds(gcopy, ocopy, nchunks, reduce_chunk):
    for b in range(_NBUF - 1):
        gcopy(b, b).start()

    def quad_body(i, _):
        k0 = i * _NBUF
        for b in range(_NBUF):
            k = k0 + b

            @pl.when(k + _NBUF - 1 < nchunks)
            def _():
                gcopy(k + _NBUF - 1, (b + _NBUF - 1) % _NBUF).start()

            gcopy(k, b).wait()

            @pl.when(k >= _NBUF)
            def _():
                ocopy(k - _NBUF, b).wait()

            reduce_chunk(b)
            ocopy(k, b).start()
        return 0

    lax.fori_loop(0, nchunks // _NBUF, quad_body, 0)
    for b in range(_NBUF):
        ocopy(nchunks - _NBUF + b, b).wait()


@functools.partial(
    pl.kernel,
    out_type=jax.ShapeDtypeStruct((N_PAD, HIDDEN), jnp.bfloat16),
    mesh=plsc.VectorSubcoreMesh(core_axis_name="c", subcore_axis_name="s"),
    compiler_params=pltpu.CompilerParams(use_tc_tiling_on_sc=False,
                                         needs_layout_passes=False),
    scratch_types=[
        pltpu.VMEM((CHUNKS, 128), jnp.int32),
        pltpu.VMEM((_NBUF, 128, HIDDEN), jnp.bfloat16),
        pltpu.VMEM((_NBUF, CHUNK_ATOMS, HIDDEN), jnp.bfloat16),
        pltpu.VMEM_SHARED((N_ATOMS, HIDDEN), jnp.bfloat16),
    ] + [pltpu.SemaphoreType.DMA] * (2 * _NBUF),
)
def _round_msg(mtab, idx_hbm, out_hbm, idx_v, rows_v, out_v, shared, *sems):
    gsems, osems = sems[:_NBUF], sems[_NBUF:]
    wid = lax.axis_index("s") * 2 + lax.axis_index("c")
    pltpu.sync_copy(idx_hbm.at[wid], idx_v)

    sub = lax.axis_index("s")
    rpw = N_ATOMS // 16
    pltpu.sync_copy(mtab.at[pl.ds(sub * rpw, rpw)],
                    shared.at[pl.ds(sub * rpw, rpw)])
    plsc.subcore_barrier()

    def gcopy(k, b):
        return pltpu.make_async_copy(
            shared.at[idx_v.at[k]], rows_v.at[b], gsems[b])

    def ocopy(k, b):
        return pltpu.make_async_copy(
            out_v.at[b],
            out_hbm.at[pl.ds(wid * APW + k * CHUNK_ATOMS, CHUNK_ATOMS)],
            osems[b])

    def reduce_chunk(b):
        for a in range(CHUNK_ATOMS):
            f32accs = [None] * (2 * _MGROUPS)
            for r in range(MAX_NEI):
                for g in range(_MGROUPS):
                    lo, hi = plsc.unpack(
                        rows_v[b, a * MAX_NEI + r, pl.ds(32 * g, 32)],
                        format=plsc.PackFormat.INTERLEAVED)
                    if f32accs[2 * g] is None:
                        f32accs[2 * g], f32accs[2 * g + 1] = lo, hi
                    else:
                        f32accs[2 * g] = f32accs[2 * g] + lo
                        f32accs[2 * g + 1] = f32accs[2 * g + 1] + hi
            for g in range(_MGROUPS):
                out_v[b, a, pl.ds(32 * g, 32)] = plsc.pack(
                    f32accs[2 * g], f32accs[2 * g + 1],
                    format=plsc.PackFormat.INTERLEAVED)

    _pipelined_rounds(gcopy, ocopy, CHUNKS, reduce_chunk)


@functools.partial(
    pl.kernel,
    out_type=jax.ShapeDtypeStruct((N_PAD, 16), jnp.float32),
    mesh=plsc.VectorSubcoreMesh(core_axis_name="c", subcore_axis_name="s"),
    compiler_params=pltpu.CompilerParams(use_tc_tiling_on_sc=False,
                                         needs_layout_passes=False),
    scratch_types=[
        pltpu.VMEM((CHUNKS, 128), jnp.int32),
        pltpu.VMEM((_NBUF, 128, 16), jnp.float32),
        pltpu.VMEM((_NBUF, CHUNK_ATOMS, 16), jnp.float32),
    ] + [pltpu.SemaphoreType.DMA] * (2 * _NBUF),
)
def _round_bond(btab, idx_hbm, out_hbm, idx_v, rows_v, out_v, *sems):
    gsems, osems = sems[:_NBUF], sems[_NBUF:]
    wid = lax.axis_index("s") * 2 + lax.axis_index("c")
    pltpu.sync_copy(idx_hbm.at[wid], idx_v)

    def gcopy(k, b):
        return pltpu.make_async_copy(
            btab.at[idx_v.at[k]], rows_v.at[b], gsems[b])

    def ocopy(k, b):
        return pltpu.make_async_copy(
            out_v.at[b],
            out_hbm.at[pl.ds(wid * APW + k * CHUNK_ATOMS, CHUNK_ATOMS)],
            osems[b])

    def reduce_chunk(b):
        for a in range(CHUNK_ATOMS):
            acc = rows_v[b, a * MAX_NEI, pl.ds(0, 16)]
            for r in range(1, MAX_NEI):
                acc = acc + rows_v[b, a * MAX_NEI + r, pl.ds(0, 16)]
            out_v[b, a, pl.ds(0, 16)] = acc

    _pipelined_rounds(gcopy, ocopy, CHUNKS, reduce_chunk)


_ROWS = 2000
_GRID = N_ATOMS // _ROWS


def _row_mask(pid, x):
    rows = lax.broadcasted_iota(jnp.int32, x.shape, 0) + pid * _ROWS
    return jnp.where(rows == 0, 0.0, x)


def _prologue_body(x_ref, wi_ref, w0_ref, w1_ref, w2_ref,
                   inp_ref, inpb_ref, h_ref):
    pid = pl.program_id(0)
    x = x_ref[...]
    inp = jnp.dot(x, wi_ref[...], preferred_element_type=jnp.float32)
    inp = _row_mask(pid, inp)
    inp_ref[...] = inp
    inpb_ref[...] = inp.astype(jnp.bfloat16)
    h = _gelu_exact(jnp.dot(x, w0_ref[...], preferred_element_type=jnp.float32))
    h = _gelu_exact(jnp.dot(h, w1_ref[...], preferred_element_type=jnp.float32))
    h = _gelu_exact(jnp.dot(h, w2_ref[...], preferred_element_type=jnp.float32))
    h_ref[...] = h


def _tc_prologue(f_atoms, W_i, W0, W1, W2):
    return pl.pallas_call(
        _prologue_body,
        grid=(_GRID,),
        in_specs=[
            pl.BlockSpec((_ROWS, ATOM_FDIM), lambda i: (i, 0)),
            pl.BlockSpec((ATOM_FDIM, HIDDEN), lambda i: (0, 0)),
            pl.BlockSpec((ATOM_FDIM, HIDDEN), lambda i: (0, 0)),
            pl.BlockSpec((HIDDEN, HIDDEN), lambda i: (0, 0)),
            pl.BlockSpec((HIDDEN, HIDDEN), lambda i: (0, 0)),
        ],
        out_specs=[
            pl.BlockSpec((_ROWS, HIDDEN), lambda i: (i, 0)),
            pl.BlockSpec((_ROWS, HIDDEN), lambda i: (i, 0)),
            pl.BlockSpec((_ROWS, HIDDEN), lambda i: (i, 0)),
        ],
        out_shape=[
            jax.ShapeDtypeStruct((N_ATOMS, HIDDEN), jnp.float32),
            jax.ShapeDtypeStruct((N_ATOMS, HIDDEN), jnp.bfloat16),
            jax.ShapeDtypeStruct((N_ATOMS, HIDDEN), jnp.float32),
        ],
    )(f_atoms, W_i, W0, W1, W2)


def _update_body(m_ref, s_ref, b_ref, wt_ref, wb_ref, o_ref, ob_ref):
    pid = pl.program_id(0)
    s = s_ref[...].astype(jnp.float32)
    m = (m_ref[...]
         + jnp.dot(s, wt_ref[...], preferred_element_type=jnp.float32)
         + jnp.dot(b_ref[...], wb_ref[...], preferred_element_type=jnp.float32))
    m = _row_mask(pid, m)
    o_ref[...] = m
    ob_ref[...] = m.astype(jnp.bfloat16)


def _tc_update(message, s, sumb, Wh_top, Wh_bot16):
    return pl.pallas_call(
        _update_body,
        grid=(_GRID,),
        in_specs=[
            pl.BlockSpec((_ROWS, HIDDEN), lambda i: (i, 0)),
            pl.BlockSpec((_ROWS, HIDDEN), lambda i: (i, 0)),
            pl.BlockSpec((_ROWS, 16), lambda i: (i, 0)),
            pl.BlockSpec((HIDDEN, HIDDEN), lambda i: (0, 0)),
            pl.BlockSpec((16, HIDDEN), lambda i: (0, 0)),
        ],
        out_specs=[
            pl.BlockSpec((_ROWS, HIDDEN), lambda i: (i, 0)),
            pl.BlockSpec((_ROWS, HIDDEN), lambda i: (i, 0)),
        ],
        out_shape=[
            jax.ShapeDtypeStruct((N_ATOMS, HIDDEN), jnp.float32),
            jax.ShapeDtypeStruct((N_ATOMS, HIDDEN), jnp.bfloat16),
        ],
    )(message, s, sumb, Wh_top, Wh_bot16)


def _final_body(h_ref, s_ref, wt_ref, wb_ref, o_ref):
    s = s_ref[...].astype(jnp.float32)
    o = (jnp.dot(h_ref[...], wt_ref[...], preferred_element_type=jnp.float32)
         + jnp.dot(s, wb_ref[...], preferred_element_type=jnp.float32))
    o_ref[...] = _gelu_exact(o)


def _tc_final(h, s, Wo_top, Wo_bot):
    return pl.pallas_call(
        _final_body,
        grid=(_GRID,),
        in_specs=[
            pl.BlockSpec((_ROWS, HIDDEN), lambda i: (i, 0)),
            pl.BlockSpec((_ROWS, HIDDEN), lambda i: (i, 0)),
            pl.BlockSpec((HIDDEN, HIDDEN), lambda i: (0, 0)),
            pl.BlockSpec((HIDDEN, HIDDEN), lambda i: (0, 0)),
        ],
        out_specs=pl.BlockSpec((_ROWS, HIDDEN), lambda i: (i, 0)),
        out_shape=jax.ShapeDtypeStruct((N_ATOMS, HIDDEN), jnp.float32),
    )(h, s, Wo_top, Wo_bot)


def _pack_idx(idx):
    idx = jnp.pad(idx.astype(jnp.int32), ((0, N_PAD - N_ATOMS), (0, 0)))
    return idx.reshape(NW, CHUNKS, 128)


def kernel(f_atoms, f_bonds, a2a, a2b, W_i, W_ah0, W_ah1, W_ah2,
           W_h0, W_h1, W_h2, W_o):
    idx_a = _pack_idx(a2a)
    idx_b = _pack_idx(a2b)
    f_bonds16 = jnp.pad(f_bonds, ((0, 0), (0, 16 - BOND_FDIM)))

    W_h = [W_h0, W_h1, W_h2]
    Wh_top = [w[:HIDDEN] for w in W_h]
    Wh_bot16 = [jnp.pad(w[HIDDEN:], ((0, 2), (0, 0))) for w in W_h]

    inp, inp_bf, h = _tc_prologue(f_atoms, W_i, W_ah0, W_ah1, W_ah2)
    sumb = _round_bond(f_bonds16, idx_b)[:N_ATOMS]

    message, message_bf = inp, inp_bf
    for d in range(DEPTH):
        s = _round_msg(message_bf, idx_a)[:N_ATOMS]
        message, message_bf = _tc_update(message, s, sumb,
                                         Wh_top[d], Wh_bot16[d])

    s = _round_msg(message_bf, idx_a)[:N_ATOMS]
    return _tc_final(h, s, W_o[:HIDDEN], W_o[HIDDEN:])
